# baseline (speedup 1.0000x reference)
import functools

import jax
import jax.numpy as jnp
from jax import lax
from jax.experimental import pallas as pl
from jax.experimental.pallas import tpu as pltpu
from jax.experimental.pallas import tpu_sc as plsc

_K = 2048
_ROWS = 128
_COLS = 32768
_LANES = 16
_NCHUNK = _COLS // _LANES
_UNROLL = 8


def _scan_hist(hist, k):
    iota = lax.iota(jnp.int32, 16)

    def cond(c):
        return c[5] == 0

    def body(c):
        j, above, b, kp, cb, _ = c
        base = 240 - j * 16
        v = hist[pl.ds(base, 16)]
        desc = lax.rev(v, (0,))
        csum = lax.cumsum(desc, axis=0) + above
        m = csum >= k
        nfound = jnp.sum(m.astype(jnp.int32))
        firstm = jnp.logical_and(m, lax.cumsum(m.astype(jnp.int32), axis=0) == 1)
        bb = jnp.sum(jnp.where(firstm, base + 15 - iota, 0))
        kk = k - jnp.sum(jnp.where(firstm, csum - desc, 0))
        cc = jnp.sum(jnp.where(firstm, desc, 0))
        tot = jnp.sum(v)
        found = (nfound > 0).astype(jnp.int32)
        return (j + 1,
                jnp.where(found == 1, above, above + tot),
                jnp.where(found == 1, bb, b),
                jnp.where(found == 1, kk, kp),
                jnp.where(found == 1, cc, cb),
                found)

    out = lax.while_loop(cond, body, (jnp.int32(0), jnp.int32(0),
                                      jnp.int32(0), k, jnp.int32(0),
                                      jnp.int32(0)))
    return out[2], out[3], out[4]


def _zero_hist(hist):
    def zbody(i):
        hist[pl.ds(i * 16, 16)] = jnp.zeros((16,), jnp.int32)
    plsc.parallel_loop(0, 16, unroll=4)(zbody)


def _make_sc_kernel():
    info = plsc.get_sparse_core_info()
    nc, ns = info.num_cores, info.num_subcores
    rows_per_tile = _ROWS // (nc * ns)
    mesh = plsc.VectorSubcoreMesh(core_axis_name="c", subcore_axis_name="s")

    @functools.partial(
        pl.kernel,
        mesh=mesh,
        out_type=jax.ShapeDtypeStruct((_ROWS, _COLS), jnp.float32),
        compiler_params=pltpu.CompilerParams(needs_layout_passes=False),
        scratch_types=[
            pltpu.VMEM((_COLS,), jnp.float32),
            pltpu.VMEM((_COLS,), jnp.float32),
            pltpu.VMEM((_COLS,), jnp.uint32),
            pltpu.VMEM((256,), jnp.int32),
            pltpu.SemaphoreType.DMA,
            pltpu.SemaphoreType.DMA,
            pltpu.SemaphoreType.DMA,
            pltpu.SemaphoreType.DMA,
        ],
    )
    def sc_topk(x_hbm, out_hbm, row_a, row_b, key_v, hist,
                sin_a, sin_b, sout_a, sout_b):
        wid = lax.axis_index("s") * nc + lax.axis_index("c")
        ones = jnp.ones((16,), jnp.int32)
        base_row = wid * rows_per_tile
        bufs = (row_a, row_b)
        sins = (sin_a, sin_b)
        souts = (sout_a, sout_b)

        def compute_row(row_v, mid_cb):
            _zero_hist(hist)

            def h1_body(i):
                v = row_v[pl.ds(i * 16, 16)]
                ui = lax.bitcast_convert_type(v, jnp.int32)
                flip = (ui >> 31).astype(jnp.uint32) | jnp.uint32(0x80000000)
                key = ui.astype(jnp.uint32) ^ flip
                key_v[pl.ds(i * 16, 16)] = key
                plsc.addupdate_scatter(hist, [(key >> jnp.uint32(24)).astype(jnp.int32)], ones)

            plsc.parallel_loop(0, _NCHUNK, unroll=_UNROLL)(h1_body)
            mid_cb()
            b1, k2, _c1 = _scan_hist(hist, jnp.int32(_K))
            pfx1 = b1.astype(jnp.uint32)

            def level_pass(shift, pfx):
                _zero_hist(hist)
                sd = jnp.uint32(shift - 8)
                base = pfx << jnp.uint32(8)

                def hbody(i):
                    key = key_v[pl.ds(i * 16, 16)]
                    rel = (key >> sd) - base
                    m = rel < jnp.uint32(256)
                    plsc.addupdate_scatter(hist, [rel.astype(jnp.int32)], ones, mask=m)

                plsc.parallel_loop(0, _NCHUNK, unroll=_UNROLL)(hbody)
                return base

            def level(shift, pfx, k):
                base = level_pass(shift, pfx)
                b, kn, c = _scan_hist(hist, k)
                return base | b.astype(jnp.uint32), kn, c

            pfx2, k3, _c2 = level(24, pfx1, k2)
            pfx3, k4, c3 = level(16, pfx2, k3)

            straddle = k4 < c3

            def _do_l4():
                level_pass(8, pfx3)

            pl.when(straddle)(_do_l4)
            b4, _kn4, _c4 = _scan_hist(hist, k4)
            thresh = (pfx3 << jnp.uint32(8)) | jnp.where(
                straddle, b4.astype(jnp.uint32), jnp.uint32(0))

            tbits = jnp.where(thresh >= jnp.uint32(0x80000000),
                              thresh ^ jnp.uint32(0x80000000),
                              ~thresh)
            tval = lax.bitcast_convert_type(tbits, jnp.float32)

            def mask_body(i):
                v = row_v[pl.ds(i * 16, 16)]
                row_v[pl.ds(i * 16, 16)] = jnp.where(
                    v >= tval, jnp.maximum(v, 0.0), 0.0)

            plsc.parallel_loop(0, _NCHUNK, unroll=_UNROLL)(mask_body)

        pltpu.async_copy(x_hbm.at[base_row], bufs[0], sins[0])
        for rr in range(rows_per_tile):
            a = rr % 2
            b = (rr + 1) % 2
            pltpu.make_async_copy(x_hbm.at[base_row + rr], bufs[a], sins[a]).wait()

            def mid_cb(rr=rr, a=a, b=b):
                if rr >= 1:
                    pltpu.make_async_copy(bufs[b], out_hbm.at[base_row + rr - 1],
                                          souts[b]).wait()
                if rr + 1 < rows_per_tile:
                    pltpu.async_copy(x_hbm.at[base_row + rr + 1], bufs[b], sins[b])

            compute_row(bufs[a], mid_cb)
            pltpu.async_copy(bufs[a], out_hbm.at[base_row + rr], souts[a])
        last = rows_per_tile - 1
        pltpu.make_async_copy(bufs[last % 2], out_hbm.at[base_row + last],
                              souts[last % 2]).wait()

    return sc_topk


_sc_kernel = None


def kernel(x):
    global _sc_kernel
    if _sc_kernel is None:
        _sc_kernel = _make_sc_kernel()
    return _sc_kernel(x)


# speedup vs baseline: 1.0902x; 1.0902x over previous
import functools

import jax
import jax.numpy as jnp
from jax import lax
from jax.experimental import pallas as pl
from jax.experimental.pallas import tpu as pltpu
from jax.experimental.pallas import tpu_sc as plsc

_K = 2048
_ROWS = 128
_COLS = 32768
_LANES = 16
_NCHUNK = _COLS // _LANES
_UNROLL = 8


def _scan_hist(hist, k):
    iota = lax.iota(jnp.int32, 16)

    def cond(c):
        return c[5] == 0

    def body(c):
        j, above, b, kp, cb, _ = c
        base = 240 - j * 16
        v = hist[pl.ds(base, 16)]
        desc = lax.rev(v, (0,))
        csum = lax.cumsum(desc, axis=0) + above
        m = csum >= k
        nfound = jnp.sum(m.astype(jnp.int32))
        firstm = jnp.logical_and(m, lax.cumsum(m.astype(jnp.int32), axis=0) == 1)
        bb = jnp.sum(jnp.where(firstm, base + 15 - iota, 0))
        kk = k - jnp.sum(jnp.where(firstm, csum - desc, 0))
        cc = jnp.sum(jnp.where(firstm, desc, 0))
        tot = jnp.sum(v)
        found = (nfound > 0).astype(jnp.int32)
        return (j + 1,
                jnp.where(found == 1, above, above + tot),
                jnp.where(found == 1, bb, b),
                jnp.where(found == 1, kk, kp),
                jnp.where(found == 1, cc, cb),
                found)

    out = lax.while_loop(cond, body, (jnp.int32(0), jnp.int32(0),
                                      jnp.int32(0), k, jnp.int32(0),
                                      jnp.int32(0)))
    return out[2], out[3], out[4]


def _zero_hist(hist):
    def zbody(i):
        hist[pl.ds(i * 16, 16)] = jnp.zeros((16,), jnp.int32)
    plsc.parallel_loop(0, 16, unroll=4)(zbody)


def _make_sc_kernel():
    info = plsc.get_sparse_core_info()
    nc, ns = info.num_cores, info.num_subcores
    rows_per_tile = _ROWS // (nc * ns)
    mesh = plsc.VectorSubcoreMesh(core_axis_name="c", subcore_axis_name="s")

    @functools.partial(
        pl.kernel,
        mesh=mesh,
        out_type=jax.ShapeDtypeStruct((_ROWS, _COLS), jnp.float32),
        compiler_params=pltpu.CompilerParams(needs_layout_passes=False),
        scratch_types=[
            pltpu.VMEM((_COLS,), jnp.float32),
            pltpu.VMEM((_COLS,), jnp.float32),
            pltpu.VMEM((_COLS,), jnp.uint32),
            pltpu.VMEM((256,), jnp.int32),
            pltpu.VMEM((256,), jnp.int32),
            pltpu.SemaphoreType.DMA,
            pltpu.SemaphoreType.DMA,
            pltpu.SemaphoreType.DMA,
            pltpu.SemaphoreType.DMA,
        ],
    )
    def sc_topk(x_hbm, out_hbm, row_a, row_b, key_v, hist, hist2,
                sin_a, sin_b, sout_a, sout_b):
        wid = lax.axis_index("s") * nc + lax.axis_index("c")
        ones = jnp.ones((16,), jnp.int32)
        base_row = wid * rows_per_tile
        bufs = (row_a, row_b)
        sins = (sin_a, sin_b)
        souts = (sout_a, sout_b)

        def compute_row(row_v, mid_cb):
            guess = jnp.uint32(0xBF)
            gbase = guess << jnp.uint32(8)
            _zero_hist(hist)
            _zero_hist(hist2)

            def h1_body(i):
                v = row_v[pl.ds(i * 16, 16)]
                ui = lax.bitcast_convert_type(v, jnp.int32)
                flip = (ui >> 31).astype(jnp.uint32) | jnp.uint32(0x80000000)
                key = ui.astype(jnp.uint32) ^ flip
                key_v[pl.ds(i * 16, 16)] = key
                plsc.addupdate_scatter(hist, [(key >> jnp.uint32(24)).astype(jnp.int32)], ones)
                rel2 = (key >> jnp.uint32(16)) - gbase
                m2 = rel2 < jnp.uint32(256)
                plsc.addupdate_scatter(hist2, [rel2.astype(jnp.int32)], ones, mask=m2)

            plsc.parallel_loop(0, _NCHUNK, unroll=_UNROLL)(h1_body)
            mid_cb()
            b1, k2, _c1 = _scan_hist(hist, jnp.int32(_K))
            pfx1 = b1.astype(jnp.uint32)

            def level_pass(href, shift, pfx):
                _zero_hist(href)
                sd = jnp.uint32(shift - 8)
                base = pfx << jnp.uint32(8)

                def hbody(i):
                    key = key_v[pl.ds(i * 16, 16)]
                    rel = (key >> sd) - base
                    m = rel < jnp.uint32(256)
                    plsc.addupdate_scatter(href, [rel.astype(jnp.int32)], ones, mask=m)

                plsc.parallel_loop(0, _NCHUNK, unroll=_UNROLL)(hbody)
                return base

            def _l2_fallback():
                level_pass(hist2, 24, pfx1)

            pl.when(pfx1 != guess)(_l2_fallback)
            b2, k3, _c2 = _scan_hist(hist2, k2)
            pfx2 = (pfx1 << jnp.uint32(8)) | b2.astype(jnp.uint32)

            base3 = level_pass(hist, 16, pfx2)
            b3, k4, c3 = _scan_hist(hist, k3)
            pfx3 = base3 | b3.astype(jnp.uint32)

            straddle = k4 < c3

            def _do_l4():
                level_pass(hist, 8, pfx3)

            pl.when(straddle)(_do_l4)
            b4, _kn4, _c4 = _scan_hist(hist, k4)
            thresh = (pfx3 << jnp.uint32(8)) | jnp.where(
                straddle, b4.astype(jnp.uint32), jnp.uint32(0))

            tbits = jnp.where(thresh >= jnp.uint32(0x80000000),
                              thresh ^ jnp.uint32(0x80000000),
                              ~thresh)
            tval = lax.bitcast_convert_type(tbits, jnp.float32)

            def mask_body(i):
                v = row_v[pl.ds(i * 16, 16)]
                row_v[pl.ds(i * 16, 16)] = jnp.where(
                    v >= tval, jnp.maximum(v, 0.0), 0.0)

            plsc.parallel_loop(0, _NCHUNK, unroll=_UNROLL)(mask_body)

        pltpu.async_copy(x_hbm.at[base_row], bufs[0], sins[0])
        for rr in range(rows_per_tile):
            a = rr % 2
            b = (rr + 1) % 2
            pltpu.make_async_copy(x_hbm.at[base_row + rr], bufs[a], sins[a]).wait()

            def mid_cb(rr=rr, a=a, b=b):
                if rr >= 1:
                    pltpu.make_async_copy(bufs[b], out_hbm.at[base_row + rr - 1],
                                          souts[b]).wait()
                if rr + 1 < rows_per_tile:
                    pltpu.async_copy(x_hbm.at[base_row + rr + 1], bufs[b], sins[b])

            compute_row(bufs[a], mid_cb)
            pltpu.async_copy(bufs[a], out_hbm.at[base_row + rr], souts[a])
        last = rows_per_tile - 1
        pltpu.make_async_copy(bufs[last % 2], out_hbm.at[base_row + last],
                              souts[last % 2]).wait()

    return sc_topk


_sc_kernel = None


def kernel(x):
    global _sc_kernel
    if _sc_kernel is None:
        _sc_kernel = _make_sc_kernel()
    return _sc_kernel(x)
